# baseline (speedup 1.0000x reference)
import functools

import jax
import jax.numpy as jnp
from jax import lax
from jax.experimental import pallas as pl
from jax.experimental.pallas import tpu as pltpu
from jax.experimental.pallas import tpu_sc as plsc

N = 10000
E = 320000
IN_CH = 128
HID = 64
NC = 2
NS = 16
NW = NC * NS
EPW = E // NW
K = 125
NCH = EPW // K
NBUF = 5
N_PAD = 10240
RPW = N_PAD // NS
ZR = 128
RB = 2000

@functools.cache
def _deg_kernel_fn():
    mesh = plsc.VectorSubcoreMesh(
        core_axis_name="c", subcore_axis_name="s", num_cores=NC)
    return pl.kernel(
        _deg_body,
        out_type=jax.ShapeDtypeStruct((NC, N_PAD, 32), jnp.bfloat16),
        mesh=mesh,
        scratch_types=[
            pltpu.VMEM((NCH, K), jnp.int32),
            pltpu.VMEM((K, 32), jnp.bfloat16),
            pltpu.VMEM((ZR, 32), jnp.bfloat16),
            pltpu.VMEM_SHARED((N_PAD, 32), jnp.bfloat16),
            pltpu.SemaphoreType.DMA,
        ],
        compiler_params=pltpu.CompilerParams(use_tc_tiling_on_sc=False),
    )


def _deg_body(ed_hbm, deg_out, dst_v, ones_v, zrow_v, deg_sh, dsem):
    cid = lax.axis_index("c")
    sid = lax.axis_index("s")
    wid = cid * NS + sid
    pltpu.sync_copy(ed_hbm.at[NW + wid], dst_v)

    def fill(r, _):
        ones_v[r, :] = jnp.ones((32,), jnp.bfloat16)
        return 0

    lax.fori_loop(0, K, fill, 0)

    def fillz(r, _):
        zrow_v[r, :] = jnp.zeros((32,), jnp.bfloat16)
        return 0

    lax.fori_loop(0, ZR, fillz, 0)

    base = sid * RPW

    def zcopy(j, _):
        pltpu.sync_copy(zrow_v, deg_sh.at[pl.ds(base + j * ZR, ZR)])
        return 0

    lax.fori_loop(0, RPW // ZR, zcopy, 0)
    plsc.subcore_barrier()

    def scat_group(i, _):
        for j in range(5):
            pltpu.async_copy(ones_v, deg_sh.at[dst_v.at[i * 5 + j]], dsem,
                             add=True)
        for j in range(5):
            pltpu.make_async_copy(ones_v, deg_sh.at[dst_v.at[i * 5 + j]],
                                  dsem).wait()
        return 0

    lax.fori_loop(0, NCH // 5, scat_group, 0)
    plsc.subcore_barrier()
    pltpu.sync_copy(deg_sh.at[pl.ds(base, RPW)], deg_out.at[cid, pl.ds(base, RPW)])


@functools.cache
def _scatter_kernel_fn():
    mesh = plsc.VectorSubcoreMesh(
        core_axis_name="c", subcore_axis_name="s", num_cores=NC)
    return pl.kernel(
        _scatter_body,
        out_type=jax.ShapeDtypeStruct((NC, N_PAD, HID), jnp.bfloat16),
        mesh=mesh,
        scratch_types=[
            pltpu.VMEM((2 * NCH, K), jnp.int32),
        ] + [pltpu.VMEM((K, HID), jnp.bfloat16) for _ in range(NBUF)]
        + [
            pltpu.VMEM_SHARED((N_PAD, HID), jnp.bfloat16),
        ] + [pltpu.SemaphoreType.DMA for _ in range(2 * NBUF)],
        compiler_params=pltpu.CompilerParams(use_tc_tiling_on_sc=False),
    )


def _scatter_body(g_hbm, ed_hbm, acc_out, ed_v, *rest):
    rows = rest[:NBUF]
    acc_sh = rest[NBUF]
    gsem = rest[NBUF + 1:NBUF + 1 + NBUF]
    ssem = rest[NBUF + 1 + NBUF:]
    cid = lax.axis_index("c")
    sid = lax.axis_index("s")
    wid = cid * NS + sid
    pltpu.sync_copy(ed_hbm.at[wid], ed_v.at[pl.ds(0, NCH)])
    pltpu.sync_copy(ed_hbm.at[NW + wid], ed_v.at[pl.ds(NCH, NCH)])

    def fillz(r, _):
        for j in range(NBUF):
            for l in range(HID // 32):
                rows[j][r, pl.ds(l * 32, 32)] = jnp.zeros((32,), jnp.bfloat16)
        return 0

    lax.fori_loop(0, K, fillz, 0)

    base = sid * RPW
    for j in range(NBUF):
        pltpu.sync_copy(rows[j], acc_sh.at[pl.ds(base + j * K, K)])
    pltpu.sync_copy(rows[0].at[pl.ds(0, RPW - NBUF * K)],
                    acc_sh.at[pl.ds(base + NBUF * K, RPW - NBUF * K)])
    plsc.subcore_barrier()

    for j in range(3):
        pltpu.async_copy(g_hbm.at[ed_v.at[j]], rows[j], gsem[j])

    def group(i, _):
        for j in range(NBUF):
            c = i * NBUF + j
            jp = (j + 3) % NBUF
            pltpu.make_async_copy(g_hbm.at[ed_v.at[c]], rows[j],
                                  gsem[j]).wait()
            pltpu.async_copy(rows[j], acc_sh.at[ed_v.at[NCH + c]], ssem[j],
                             add=True)

            @pl.when(c >= 2)
            def _():
                pltpu.make_async_copy(rows[jp], acc_sh.at[ed_v.at[NCH + c - 2]],
                                      ssem[jp]).wait()

            @pl.when(c + 3 < NCH)
            def _():
                pltpu.async_copy(g_hbm.at[ed_v.at[c + 3]], rows[jp], gsem[jp])
        return 0

    lax.fori_loop(0, NCH // NBUF, group, 0)
    pltpu.make_async_copy(rows[NBUF - 2], acc_sh.at[ed_v.at[2 * NCH - 2]],
                          ssem[NBUF - 2]).wait()
    pltpu.make_async_copy(rows[NBUF - 1], acc_sh.at[ed_v.at[2 * NCH - 1]],
                          ssem[NBUF - 1]).wait()
    plsc.subcore_barrier()
    pltpu.sync_copy(acc_sh.at[pl.ds(base, RPW)], acc_out.at[cid, pl.ds(base, RPW)])


def _lin_body(x_ref, w_ref, degp_ref, g_ref):
    deg = (degp_ref[0, :, 0] + degp_ref[1, :, 0]).astype(jnp.float32) + 1.0
    dinv = lax.rsqrt(deg)
    h = jnp.dot(x_ref[...], w_ref[...], preferred_element_type=jnp.float32)
    g_ref[...] = (h * dinv[:, None]).astype(jnp.bfloat16)


def _lin(x, W, degp):
    return pl.pallas_call(
        _lin_body,
        grid=(N // RB,),
        in_specs=[
            pl.BlockSpec((RB, IN_CH), lambda i: (i, 0)),
            pl.BlockSpec((IN_CH, HID), lambda i: (0, 0)),
            pl.BlockSpec((NC, RB, 32), lambda i: (0, i, 0)),
        ],
        out_specs=pl.BlockSpec((RB, HID), lambda i: (i, 0)),
        out_shape=jax.ShapeDtypeStruct((N, HID), jnp.bfloat16),
    )(x, W, degp)


def _tail_body(acc_ref, g_ref, degp_ref, b_ref, out_ref):
    deg = (degp_ref[0, :, 0] + degp_ref[1, :, 0]).astype(jnp.float32) + 1.0
    dinv = lax.rsqrt(deg)
    acc = acc_ref[0].astype(jnp.float32) + acc_ref[1].astype(jnp.float32)
    s = (acc + g_ref[...].astype(jnp.float32)) * dinv[:, None] + b_ref[0]
    s = jnp.maximum(s, 0.0)
    m = jnp.max(s, axis=1, keepdims=True)
    lse = jnp.log(jnp.sum(jnp.exp(s - m), axis=1, keepdims=True)) + m
    out_ref[...] = s - lse


def _tail(acc, g, degp, b2):
    return pl.pallas_call(
        _tail_body,
        grid=(N // RB,),
        in_specs=[
            pl.BlockSpec((NC, RB, HID), lambda i: (0, i, 0)),
            pl.BlockSpec((RB, HID), lambda i: (i, 0)),
            pl.BlockSpec((NC, RB, 32), lambda i: (0, i, 0)),
            pl.BlockSpec((1, HID), lambda i: (0, 0)),
        ],
        out_specs=pl.BlockSpec((RB, HID), lambda i: (i, 0)),
        out_shape=jax.ShapeDtypeStruct((N, HID), jnp.float32),
    )(acc, g, degp, b2)


@jax.jit
def kernel(x, edge_index, W, b):
    ed = edge_index.astype(jnp.int32).reshape(2 * NW, NCH, K)
    degp = _deg_kernel_fn()(ed)
    g = _lin(x, W, degp)
    acc = _scatter_kernel_fn()(g, ed)
    return _tail(acc, g, degp, b.reshape(1, HID))


# speedup vs baseline: 1.0007x; 1.0007x over previous
import functools

import jax
import jax.numpy as jnp
from jax import lax
from jax.experimental import pallas as pl
from jax.experimental.pallas import tpu as pltpu
from jax.experimental.pallas import tpu_sc as plsc

N = 10000
E = 320000
IN_CH = 128
HID = 64
NC = 2
NS = 16
NW = NC * NS
EPW = E // NW
K = 125
NCH = EPW // K
NBUF = 5
N_PAD = 10240
RPW = N_PAD // NS
ZR = 128
RB = 2000

@functools.cache
def _deg_kernel_fn():
    mesh = plsc.VectorSubcoreMesh(
        core_axis_name="c", subcore_axis_name="s", num_cores=NC)
    return pl.kernel(
        _deg_body,
        out_type=jax.ShapeDtypeStruct((NC, N_PAD, 16), jnp.float32),
        mesh=mesh,
        scratch_types=[
            pltpu.VMEM((NCH, K), jnp.int32),
            pltpu.VMEM((K, 16), jnp.float32),
            pltpu.VMEM((ZR, 16), jnp.float32),
            pltpu.VMEM_SHARED((N_PAD, 16), jnp.float32),
            pltpu.SemaphoreType.DMA,
        ],
        compiler_params=pltpu.CompilerParams(use_tc_tiling_on_sc=False),
    )


def _deg_body(ed_hbm, deg_out, dst_v, ones_v, zrow_v, deg_sh, dsem):
    cid = lax.axis_index("c")
    sid = lax.axis_index("s")
    wid = cid * NS + sid
    pltpu.sync_copy(ed_hbm.at[NW + wid], dst_v)

    def fill(r, _):
        ones_v[r, :] = jnp.ones((16,), jnp.float32)
        return 0

    lax.fori_loop(0, K, fill, 0)

    def fillz(r, _):
        zrow_v[r, :] = jnp.zeros((16,), jnp.float32)
        return 0

    lax.fori_loop(0, ZR, fillz, 0)

    base = sid * RPW

    def zcopy(j, _):
        pltpu.sync_copy(zrow_v, deg_sh.at[pl.ds(base + j * ZR, ZR)])
        return 0

    lax.fori_loop(0, RPW // ZR, zcopy, 0)
    plsc.subcore_barrier()

    def scat_group(i, _):
        for j in range(5):
            pltpu.async_copy(ones_v, deg_sh.at[dst_v.at[i * 5 + j]], dsem,
                             add=True)
        for j in range(5):
            pltpu.make_async_copy(ones_v, deg_sh.at[dst_v.at[i * 5 + j]],
                                  dsem).wait()
        return 0

    lax.fori_loop(0, NCH // 5, scat_group, 0)
    plsc.subcore_barrier()
    pltpu.sync_copy(deg_sh.at[pl.ds(base, RPW)], deg_out.at[cid, pl.ds(base, RPW)])


@functools.cache
def _scatter_kernel_fn():
    mesh = plsc.VectorSubcoreMesh(
        core_axis_name="c", subcore_axis_name="s", num_cores=NC)
    return pl.kernel(
        _scatter_body,
        out_type=jax.ShapeDtypeStruct((NC, N_PAD, HID), jnp.bfloat16),
        mesh=mesh,
        scratch_types=[
            pltpu.VMEM((2 * NCH, K), jnp.int32),
        ] + [pltpu.VMEM((K, HID), jnp.bfloat16) for _ in range(NBUF)]
        + [
            pltpu.VMEM_SHARED((N_PAD, HID), jnp.bfloat16),
        ] + [pltpu.SemaphoreType.DMA for _ in range(2 * NBUF)],
        compiler_params=pltpu.CompilerParams(use_tc_tiling_on_sc=False),
    )


def _scatter_body(g_hbm, ed_hbm, acc_out, ed_v, *rest):
    rows = rest[:NBUF]
    acc_sh = rest[NBUF]
    gsem = rest[NBUF + 1:NBUF + 1 + NBUF]
    ssem = rest[NBUF + 1 + NBUF:]
    cid = lax.axis_index("c")
    sid = lax.axis_index("s")
    wid = cid * NS + sid
    pltpu.sync_copy(ed_hbm.at[wid], ed_v.at[pl.ds(0, NCH)])
    pltpu.sync_copy(ed_hbm.at[NW + wid], ed_v.at[pl.ds(NCH, NCH)])

    def fillz(r, _):
        for j in range(NBUF):
            for l in range(HID // 32):
                rows[j][r, pl.ds(l * 32, 32)] = jnp.zeros((32,), jnp.bfloat16)
        return 0

    lax.fori_loop(0, K, fillz, 0)

    base = sid * RPW
    for j in range(NBUF):
        pltpu.sync_copy(rows[j], acc_sh.at[pl.ds(base + j * K, K)])
    pltpu.sync_copy(rows[0].at[pl.ds(0, RPW - NBUF * K)],
                    acc_sh.at[pl.ds(base + NBUF * K, RPW - NBUF * K)])
    plsc.subcore_barrier()

    for j in range(3):
        pltpu.async_copy(g_hbm.at[ed_v.at[j]], rows[j], gsem[j])

    def group(i, _):
        for j in range(NBUF):
            c = i * NBUF + j
            jp = (j + 3) % NBUF
            pltpu.make_async_copy(g_hbm.at[ed_v.at[c]], rows[j],
                                  gsem[j]).wait()
            pltpu.async_copy(rows[j], acc_sh.at[ed_v.at[NCH + c]], ssem[j],
                             add=True)

            @pl.when(c >= 2)
            def _():
                pltpu.make_async_copy(rows[jp], acc_sh.at[ed_v.at[NCH + c - 2]],
                                      ssem[jp]).wait()

            @pl.when(c + 3 < NCH)
            def _():
                pltpu.async_copy(g_hbm.at[ed_v.at[c + 3]], rows[jp], gsem[jp])
        return 0

    lax.fori_loop(0, NCH // NBUF, group, 0)
    pltpu.make_async_copy(rows[NBUF - 2], acc_sh.at[ed_v.at[2 * NCH - 2]],
                          ssem[NBUF - 2]).wait()
    pltpu.make_async_copy(rows[NBUF - 1], acc_sh.at[ed_v.at[2 * NCH - 1]],
                          ssem[NBUF - 1]).wait()
    plsc.subcore_barrier()
    pltpu.sync_copy(acc_sh.at[pl.ds(base, RPW)], acc_out.at[cid, pl.ds(base, RPW)])


def _lin_body(x_ref, w_ref, degp_ref, g_ref):
    deg = degp_ref[0, :, 0] + degp_ref[1, :, 0] + 1.0
    dinv = lax.rsqrt(deg)
    h = jnp.dot(x_ref[...], w_ref[...], preferred_element_type=jnp.float32)
    g_ref[...] = (h * dinv[:, None]).astype(jnp.bfloat16)


def _lin(x, W, degp):
    return pl.pallas_call(
        _lin_body,
        grid=(N // RB,),
        in_specs=[
            pl.BlockSpec((RB, IN_CH), lambda i: (i, 0)),
            pl.BlockSpec((IN_CH, HID), lambda i: (0, 0)),
            pl.BlockSpec((NC, RB, 16), lambda i: (0, i, 0)),
        ],
        out_specs=pl.BlockSpec((RB, HID), lambda i: (i, 0)),
        out_shape=jax.ShapeDtypeStruct((N, HID), jnp.bfloat16),
    )(x, W, degp)


def _tail_body(acc_ref, g_ref, degp_ref, b_ref, out_ref):
    deg = degp_ref[0, :, 0] + degp_ref[1, :, 0] + 1.0
    dinv = lax.rsqrt(deg)
    acc = acc_ref[0].astype(jnp.float32) + acc_ref[1].astype(jnp.float32)
    s = (acc + g_ref[...].astype(jnp.float32)) * dinv[:, None] + b_ref[0]
    s = jnp.maximum(s, 0.0)
    m = jnp.max(s, axis=1, keepdims=True)
    lse = jnp.log(jnp.sum(jnp.exp(s - m), axis=1, keepdims=True)) + m
    out_ref[...] = s - lse


def _tail(acc, g, degp, b2):
    return pl.pallas_call(
        _tail_body,
        grid=(N // RB,),
        in_specs=[
            pl.BlockSpec((NC, RB, HID), lambda i: (0, i, 0)),
            pl.BlockSpec((RB, HID), lambda i: (i, 0)),
            pl.BlockSpec((NC, RB, 16), lambda i: (0, i, 0)),
            pl.BlockSpec((1, HID), lambda i: (0, 0)),
        ],
        out_specs=pl.BlockSpec((RB, HID), lambda i: (i, 0)),
        out_shape=jax.ShapeDtypeStruct((N, HID), jnp.float32),
    )(acc, g, degp, b2)


@jax.jit
def kernel(x, edge_index, W, b):
    ed = edge_index.astype(jnp.int32).reshape(2 * NW, NCH, K)
    degp = _deg_kernel_fn()(ed)
    g = _lin(x, W, degp)
    acc = _scatter_kernel_fn()(g, ed)
    return _tail(acc, g, degp, b.reshape(1, HID))
